# baseline (speedup 1.0000x reference)
import math

import jax
import jax.numpy as jnp
from jax.experimental import pallas as pl
from jax.experimental.pallas import tpu as pltpu

K = 1000
S = 8
D = 1024
B = 4096
BETA = 5.0
ALPHA = 2.0

_XSCALE = BETA * math.log2(math.e)
_OSCALE = ALPHA * math.exp(-BETA)

_BB = 2048


def _tda_kernel(x_ref, m_ref, o_ref, xb_ref):
    s = pl.program_id(1)

    @pl.when(s == 0)
    def _cast_x():
        xb_ref[...] = (x_ref[...] * _XSCALE).astype(jnp.bfloat16)

    mb = m_ref[...].astype(jnp.bfloat16)
    a = jax.lax.dot_general(
        xb_ref[...], mb,
        dimension_numbers=(((1,), (1,)), ((), ())),
        preferred_element_type=jnp.float32,
    )
    e = jnp.exp2(a)

    @pl.when(s == 0)
    def _init():
        o_ref[...] = e

    @pl.when((s != 0) & (s != S - 1))
    def _acc():
        o_ref[...] += e

    @pl.when(s == S - 1)
    def _fin():
        o_ref[...] = (o_ref[...] + e) * _OSCALE


def kernel(x, memory):
    m2 = memory.reshape(K, S * D)
    grid = (B // _BB, S)
    return pl.pallas_call(
        _tda_kernel,
        grid=grid,
        in_specs=[
            pl.BlockSpec((_BB, D), lambda i, s: (i, 0)),
            pl.BlockSpec((K, D), lambda i, s: (0, s)),
        ],
        out_specs=pl.BlockSpec((_BB, K), lambda i, s: (i, 0)),
        out_shape=jax.ShapeDtypeStruct((B, K), jnp.float32),
        scratch_shapes=[pltpu.VMEM((_BB, D), jnp.bfloat16)],
    )(x, m2)
